# Initial kernel scaffold; baseline (speedup 1.0000x reference)
import jax
import jax.numpy as jnp
from jax.experimental import pallas as pl


def kernel(x, edge_index, edge_attr, W1f, b1f, W2f, b2f, W1, b1, W2, b2, gamma_i, beta_i, gamma, beta):
    raise NotImplementedError("write your pallas kernel here")


# speedup vs baseline: 2.4303x; 2.4303x over previous
import functools

import jax
import jax.numpy as jnp
from jax import lax
from jax.experimental import pallas as pl
from jax.experimental.pallas import tpu as pltpu
from jax.experimental.pallas import tpu_sc as plsc

N = 10000
E = 320000
D = 128

NC = 2
NS = 16
NW = NC * NS
EPW = E // NW
CH = 128
NFULL = (EPW // CH) * CH
TAIL = EPW - NFULL

BE = 1280
GB = E // BE
EPS = 1e-5


def _vmesh():
    return plsc.VectorSubcoreMesh(core_axis_name="c", subcore_axis_name="s",
                                  num_cores=NC, num_subcores=NS)


def _sc_gather(x, src, dst):
    @functools.partial(
        pl.kernel,
        out_type=(jax.ShapeDtypeStruct((E, D), jnp.float32),
                  jax.ShapeDtypeStruct((E, D), jnp.float32)),
        mesh=_vmesh(),
        scratch_types=[
            pltpu.VMEM((CH,), jnp.int32), pltpu.VMEM((CH,), jnp.int32),
            pltpu.VMEM((CH, D), jnp.float32), pltpu.VMEM((CH, D), jnp.float32),
            pltpu.VMEM((TAIL,), jnp.int32), pltpu.VMEM((TAIL,), jnp.int32),
            pltpu.VMEM((TAIL, D), jnp.float32), pltpu.VMEM((TAIL, D), jnp.float32),
            pltpu.SemaphoreType.DMA, pltpu.SemaphoreType.DMA,
        ],
    )
    def k(x_hbm, src_hbm, dst_hbm, xs_hbm, xd_hbm,
          si, di, srows, drows, sit, dit, srowst, drowst, sem_s, sem_d):
        wid = lax.axis_index("s") * NC + lax.axis_index("c")
        base = wid * EPW

        @pl.loop(0, NFULL, step=CH)
        def _(off):
            b = base + off
            pltpu.sync_copy(src_hbm.at[pl.ds(b, CH)], si)
            pltpu.sync_copy(dst_hbm.at[pl.ds(b, CH)], di)
            cs = pltpu.async_copy(x_hbm.at[si], srows, sem_s)
            cd = pltpu.async_copy(x_hbm.at[di], drows, sem_d)
            cs.wait()
            cd.wait()
            pltpu.sync_copy(srows, xs_hbm.at[pl.ds(b, CH)])
            pltpu.sync_copy(drows, xd_hbm.at[pl.ds(b, CH)])

        bt = base + NFULL
        pltpu.sync_copy(src_hbm.at[pl.ds(bt, TAIL)], sit)
        pltpu.sync_copy(dst_hbm.at[pl.ds(bt, TAIL)], dit)
        cs = pltpu.async_copy(x_hbm.at[sit], srowst, sem_s)
        cd = pltpu.async_copy(x_hbm.at[dit], drowst, sem_d)
        cs.wait()
        cd.wait()
        pltpu.sync_copy(srowst, xs_hbm.at[pl.ds(bt, TAIL)])
        pltpu.sync_copy(drowst, xd_hbm.at[pl.ds(bt, TAIL)])

    return k(x, src, dst)


def _silu(v):
    return v * jax.nn.sigmoid(v)


def _mlp_body(xd_ref, xs_ref, ea_ref, w1f_ref, w2f_ref, w1_ref, w2_ref,
              b1f_ref, b2f_ref, b1_ref, b2_ref, hf_ref, h_ref, st_ref):
    i = pl.program_id(0)
    xd = xd_ref[...]
    xs = xs_ref[...]
    ea = ea_ref[...]
    w1f = w1f_ref[...]
    uf = (jnp.dot(xd, w1f[0:D], preferred_element_type=jnp.float32)
          + jnp.dot(xs, w1f[D:2 * D], preferred_element_type=jnp.float32)
          + jnp.dot(ea, w1f[2 * D:3 * D], preferred_element_type=jnp.float32)
          + b1f_ref[...])
    hf = jnp.dot(_silu(uf), w2f_ref[...], preferred_element_type=jnp.float32) + b2f_ref[...]
    w1 = w1_ref[...]
    u = (jnp.dot(xd, w1[0:D], preferred_element_type=jnp.float32)
         + jnp.dot(xs, w1[D:2 * D], preferred_element_type=jnp.float32)
         + jnp.dot(ea, w1[2 * D:3 * D], preferred_element_type=jnp.float32)
         + b1_ref[...])
    h = jnp.dot(_silu(u), w2_ref[...], preferred_element_type=jnp.float32) + b2_ref[...]
    hf_ref[...] = hf
    h_ref[...] = h

    s1 = jnp.sum(hf, axis=0, keepdims=True)
    s2 = jnp.sum(hf * hf, axis=0, keepdims=True)
    upd = jnp.concatenate([s1, s2, jnp.zeros((6, D), jnp.float32)], axis=0)

    @pl.when(i == 0)
    def _():
        st_ref[...] = jnp.zeros_like(st_ref)

    st_ref[...] += upd


def _tc_mlp(xd, xs, ea, w1ft, w2ft, w1t, w2t, b1f, b2f, b1, b2):
    blk = lambda: pl.BlockSpec((BE, D), lambda i: (i, 0))
    full = lambda r: pl.BlockSpec((r, D), lambda i: (0, 0))
    return pl.pallas_call(
        _mlp_body,
        grid=(GB,),
        in_specs=[blk(), blk(), blk(),
                  full(3 * D), full(D), full(3 * D), full(D),
                  full(1), full(1), full(1), full(1)],
        out_specs=[blk(), blk(), full(8)],
        out_shape=(jax.ShapeDtypeStruct((E, D), jnp.float32),
                   jax.ShapeDtypeStruct((E, D), jnp.float32),
                   jax.ShapeDtypeStruct((8, D), jnp.float32)),
    )(xd, xs, ea, w1ft, w2ft, w1t, w2t, b1f, b2f, b1, b2)


def _msg_body(hf_ref, h_ref, st_ref, gi_ref, bi_ref, msg_ref):
    st = st_ref[...]
    mean = st[0:1] * (1.0 / E)
    var = st[1:2] * (1.0 / E) - mean * mean
    inv = lax.rsqrt(var + EPS)
    score = jax.nn.sigmoid((hf_ref[...] - mean) * inv * gi_ref[...] + bi_ref[...])
    msg_ref[...] = score * h_ref[...]


def _tc_msg(hf, h, st, gi, bi):
    blk = lambda: pl.BlockSpec((BE, D), lambda i: (i, 0))
    full = lambda r: pl.BlockSpec((r, D), lambda i: (0, 0))
    return pl.pallas_call(
        _msg_body,
        grid=(GB,),
        in_specs=[blk(), blk(), full(8), full(1), full(1)],
        out_specs=blk(),
        out_shape=jax.ShapeDtypeStruct((E, D), jnp.float32),
    )(hf, h, st, gi, bi)


def _sc_scatter(msg, dst, zeros):
    RPC = 624
    RTAIL = N - RPC * NS

    @functools.partial(
        pl.kernel,
        out_type=(jax.ShapeDtypeStruct((N, D), jnp.float32),
                  jax.ShapeDtypeStruct((N, D), jnp.float32)),
        mesh=_vmesh(),
        scratch_types=[
            pltpu.VMEM((CH,), jnp.int32), pltpu.VMEM((CH, D), jnp.float32),
            pltpu.VMEM((TAIL,), jnp.int32), pltpu.VMEM((TAIL, D), jnp.float32),
            pltpu.VMEM_SHARED((N, D), jnp.float32),
        ],
    )
    def k(msg_hbm, dst_hbm, z_hbm, o0_hbm, o1_hbm, idx, rows, idxt, rowst, acc):
        cid = lax.axis_index("c")
        sid = lax.axis_index("s")
        wid = sid * NC + cid
        base = wid * EPW

        pltpu.sync_copy(z_hbm.at[pl.ds(sid * RPC, RPC)], acc.at[pl.ds(sid * RPC, RPC)])

        @pl.when(sid == NS - 1)
        def _():
            pltpu.sync_copy(z_hbm.at[pl.ds(NS * RPC, RTAIL)], acc.at[pl.ds(NS * RPC, RTAIL)])

        plsc.subcore_barrier()

        @pl.loop(0, NFULL, step=CH)
        def _(off):
            b = base + off
            pltpu.sync_copy(dst_hbm.at[pl.ds(b, CH)], idx)
            pltpu.sync_copy(msg_hbm.at[pl.ds(b, CH)], rows)
            pltpu.sync_copy(rows, acc.at[idx], add=True)

        bt = base + NFULL
        pltpu.sync_copy(dst_hbm.at[pl.ds(bt, TAIL)], idxt)
        pltpu.sync_copy(msg_hbm.at[pl.ds(bt, TAIL)], rowst)
        pltpu.sync_copy(rowst, acc.at[idxt], add=True)

        plsc.subcore_barrier()

        @pl.when(cid == 0)
        def _():
            pltpu.sync_copy(acc.at[pl.ds(sid * RPC, RPC)], o0_hbm.at[pl.ds(sid * RPC, RPC)])

            @pl.when(sid == NS - 1)
            def _():
                pltpu.sync_copy(acc.at[pl.ds(NS * RPC, RTAIL)], o0_hbm.at[pl.ds(NS * RPC, RTAIL)])

        @pl.when(cid == 1)
        def _():
            pltpu.sync_copy(acc.at[pl.ds(sid * RPC, RPC)], o1_hbm.at[pl.ds(sid * RPC, RPC)])

            @pl.when(sid == NS - 1)
            def _():
                pltpu.sync_copy(acc.at[pl.ds(NS * RPC, RTAIL)], o1_hbm.at[pl.ds(NS * RPC, RTAIL)])

    return k(msg, dst, zeros)


def _final_body(o0_ref, o1_ref, x_ref, g_ref, b_ref, y_ref):
    o = o0_ref[...] + o1_ref[...]
    mean = jnp.mean(o, axis=0, keepdims=True)
    var = jnp.mean(o * o, axis=0, keepdims=True) - mean * mean
    inv = lax.rsqrt(var + EPS)
    y = x_ref[...] + (o - mean) * inv * g_ref[...] + b_ref[...]
    y_ref[...] = jnp.maximum(y, 0.0)


def _tc_final(o0, o1, x, g, b):
    full = lambda r: pl.BlockSpec((r, D), lambda: (0, 0))
    return pl.pallas_call(
        _final_body,
        in_specs=[full(N), full(N), full(N), full(1), full(1)],
        out_specs=full(N),
        out_shape=jax.ShapeDtypeStruct((N, D), jnp.float32),
    )(o0, o1, x, g, b)


def kernel(x, edge_index, edge_attr, W1f, b1f, W2f, b2f, W1, b1, W2, b2,
           gamma_i, beta_i, gamma, beta):
    src = edge_index[0]
    dst = edge_index[1]
    xs_g, xd_g = _sc_gather(x, src, dst)
    hf, h, st = _tc_mlp(xd_g, xs_g, edge_attr,
                        W1f.T, W2f.T, W1.T, W2.T,
                        b1f[None, :], b2f[None, :], b1[None, :], b2[None, :])
    msg = _tc_msg(hf, h, st, gamma_i[None, :], beta_i[None, :])
    zeros = jnp.zeros((N, D), jnp.float32)
    o0, o1 = _sc_scatter(msg, dst, zeros)
    return _tc_final(o0, o1, x, gamma[None, :], beta[None, :])


# speedup vs baseline: 2.5538x; 1.0508x over previous
import functools

import jax
import jax.numpy as jnp
from jax import lax
from jax.experimental import pallas as pl
from jax.experimental.pallas import tpu as pltpu
from jax.experimental.pallas import tpu_sc as plsc

N = 10000
E = 320000
D = 128

NC = 2
NS = 16
NW = NC * NS
EPW = E // NW
CH = 128
NFULL = (EPW // CH) * CH
TAIL = EPW - NFULL

BE = 1280
GB = E // BE
EPS = 1e-5


def _vmesh():
    return plsc.VectorSubcoreMesh(core_axis_name="c", subcore_axis_name="s",
                                  num_cores=NC, num_subcores=NS)


def _sc_gather(x, src, dst):
    @functools.partial(
        pl.kernel,
        out_type=(jax.ShapeDtypeStruct((E, D), jnp.float32),
                  jax.ShapeDtypeStruct((E, D), jnp.float32)),
        mesh=_vmesh(),
        scratch_types=[
            pltpu.VMEM((CH,), jnp.int32), pltpu.VMEM((CH,), jnp.int32),
            pltpu.VMEM((CH, D), jnp.float32), pltpu.VMEM((CH, D), jnp.float32),
            pltpu.VMEM((TAIL,), jnp.int32), pltpu.VMEM((TAIL,), jnp.int32),
            pltpu.VMEM((TAIL, D), jnp.float32), pltpu.VMEM((TAIL, D), jnp.float32),
            pltpu.SemaphoreType.DMA, pltpu.SemaphoreType.DMA,
        ],
    )
    def k(x_hbm, src_hbm, dst_hbm, xs_hbm, xd_hbm,
          si, di, srows, drows, sit, dit, srowst, drowst, sem_s, sem_d):
        wid = lax.axis_index("s") * NC + lax.axis_index("c")
        base = wid * EPW

        @pl.loop(0, NFULL, step=CH)
        def _(off):
            b = base + off
            pltpu.sync_copy(src_hbm.at[pl.ds(b, CH)], si)
            pltpu.sync_copy(dst_hbm.at[pl.ds(b, CH)], di)
            cs = pltpu.async_copy(x_hbm.at[si], srows, sem_s)
            cd = pltpu.async_copy(x_hbm.at[di], drows, sem_d)
            cs.wait()
            cd.wait()
            pltpu.sync_copy(srows, xs_hbm.at[pl.ds(b, CH)])
            pltpu.sync_copy(drows, xd_hbm.at[pl.ds(b, CH)])

        bt = base + NFULL
        pltpu.sync_copy(src_hbm.at[pl.ds(bt, TAIL)], sit)
        pltpu.sync_copy(dst_hbm.at[pl.ds(bt, TAIL)], dit)
        cs = pltpu.async_copy(x_hbm.at[sit], srowst, sem_s)
        cd = pltpu.async_copy(x_hbm.at[dit], drowst, sem_d)
        cs.wait()
        cd.wait()
        pltpu.sync_copy(srowst, xs_hbm.at[pl.ds(bt, TAIL)])
        pltpu.sync_copy(drowst, xd_hbm.at[pl.ds(bt, TAIL)])

    return k(x, src, dst)


def _silu(v):
    return v * jax.nn.sigmoid(v)


def _mlp_body(xd_ref, xs_ref, ea_ref, w1f_ref, w2f_ref, w1_ref, w2_ref,
              b1f_ref, b2f_ref, b1_ref, b2_ref, hf_ref, h_ref, st_ref):
    i = pl.program_id(0)
    xd = xd_ref[...].astype(jnp.bfloat16)
    xs = xs_ref[...].astype(jnp.bfloat16)
    ea = ea_ref[...].astype(jnp.bfloat16)
    w1f = w1f_ref[...]
    uf = (jnp.dot(xd, w1f[0:D], preferred_element_type=jnp.float32)
          + jnp.dot(xs, w1f[D:2 * D], preferred_element_type=jnp.float32)
          + jnp.dot(ea, w1f[2 * D:3 * D], preferred_element_type=jnp.float32)
          + b1f_ref[...])
    hf = jnp.dot(_silu(uf).astype(jnp.bfloat16), w2f_ref[...],
                 preferred_element_type=jnp.float32) + b2f_ref[...]
    w1 = w1_ref[...]
    u = (jnp.dot(xd, w1[0:D], preferred_element_type=jnp.float32)
         + jnp.dot(xs, w1[D:2 * D], preferred_element_type=jnp.float32)
         + jnp.dot(ea, w1[2 * D:3 * D], preferred_element_type=jnp.float32)
         + b1_ref[...])
    h = jnp.dot(_silu(u).astype(jnp.bfloat16), w2_ref[...],
                preferred_element_type=jnp.float32) + b2_ref[...]
    hf_ref[...] = hf.astype(jnp.bfloat16)
    h_ref[...] = h.astype(jnp.bfloat16)

    s1 = jnp.sum(hf, axis=0, keepdims=True)
    s2 = jnp.sum(hf * hf, axis=0, keepdims=True)
    upd = jnp.concatenate([s1, s2, jnp.zeros((6, D), jnp.float32)], axis=0)

    @pl.when(i == 0)
    def _():
        st_ref[...] = jnp.zeros_like(st_ref)

    st_ref[...] += upd


def _tc_mlp(xd, xs, ea, w1ft, w2ft, w1t, w2t, b1f, b2f, b1, b2):
    blk = lambda: pl.BlockSpec((BE, D), lambda i: (i, 0))
    full = lambda r: pl.BlockSpec((r, D), lambda i: (0, 0))
    return pl.pallas_call(
        _mlp_body,
        grid=(GB,),
        in_specs=[blk(), blk(), blk(),
                  full(3 * D), full(D), full(3 * D), full(D),
                  full(1), full(1), full(1), full(1)],
        out_specs=[blk(), blk(), full(8)],
        out_shape=(jax.ShapeDtypeStruct((E, D), jnp.bfloat16),
                   jax.ShapeDtypeStruct((E, D), jnp.bfloat16),
                   jax.ShapeDtypeStruct((8, D), jnp.float32)),
    )(xd, xs, ea, w1ft, w2ft, w1t, w2t, b1f, b2f, b1, b2)


def _msg_body(hf_ref, h_ref, st_ref, gi_ref, bi_ref, msg_ref):
    st = st_ref[...]
    mean = st[0:1] * (1.0 / E)
    var = st[1:2] * (1.0 / E) - mean * mean
    inv = lax.rsqrt(var + EPS)
    hf = hf_ref[...].astype(jnp.float32)
    score = jax.nn.sigmoid((hf - mean) * inv * gi_ref[...] + bi_ref[...])
    msg_ref[...] = score * h_ref[...].astype(jnp.float32)


def _tc_msg(hf, h, st, gi, bi):
    blk = lambda: pl.BlockSpec((BE, D), lambda i: (i, 0))
    full = lambda r: pl.BlockSpec((r, D), lambda i: (0, 0))
    return pl.pallas_call(
        _msg_body,
        grid=(GB,),
        in_specs=[blk(), blk(), full(8), full(1), full(1)],
        out_specs=blk(),
        out_shape=jax.ShapeDtypeStruct((E, D), jnp.float32),
    )(hf, h, st, gi, bi)


def _sc_scatter(msg, dst, zeros):
    RPC = 624
    RTAIL = N - RPC * NS

    @functools.partial(
        pl.kernel,
        out_type=(jax.ShapeDtypeStruct((N, D), jnp.float32),
                  jax.ShapeDtypeStruct((N, D), jnp.float32)),
        mesh=_vmesh(),
        scratch_types=[
            pltpu.VMEM((CH,), jnp.int32), pltpu.VMEM((CH, D), jnp.float32),
            pltpu.VMEM((TAIL,), jnp.int32), pltpu.VMEM((TAIL, D), jnp.float32),
            pltpu.VMEM_SHARED((N, D), jnp.float32),
        ],
    )
    def k(msg_hbm, dst_hbm, z_hbm, o0_hbm, o1_hbm, idx, rows, idxt, rowst, acc):
        cid = lax.axis_index("c")
        sid = lax.axis_index("s")
        wid = sid * NC + cid
        base = wid * EPW

        pltpu.sync_copy(z_hbm.at[pl.ds(sid * RPC, RPC)], acc.at[pl.ds(sid * RPC, RPC)])

        @pl.when(sid == NS - 1)
        def _():
            pltpu.sync_copy(z_hbm.at[pl.ds(NS * RPC, RTAIL)], acc.at[pl.ds(NS * RPC, RTAIL)])

        plsc.subcore_barrier()

        @pl.loop(0, NFULL, step=CH)
        def _(off):
            b = base + off
            pltpu.sync_copy(dst_hbm.at[pl.ds(b, CH)], idx)
            pltpu.sync_copy(msg_hbm.at[pl.ds(b, CH)], rows)
            pltpu.sync_copy(rows, acc.at[idx], add=True)

        bt = base + NFULL
        pltpu.sync_copy(dst_hbm.at[pl.ds(bt, TAIL)], idxt)
        pltpu.sync_copy(msg_hbm.at[pl.ds(bt, TAIL)], rowst)
        pltpu.sync_copy(rowst, acc.at[idxt], add=True)

        plsc.subcore_barrier()

        @pl.when(cid == 0)
        def _():
            pltpu.sync_copy(acc.at[pl.ds(sid * RPC, RPC)], o0_hbm.at[pl.ds(sid * RPC, RPC)])

            @pl.when(sid == NS - 1)
            def _():
                pltpu.sync_copy(acc.at[pl.ds(NS * RPC, RTAIL)], o0_hbm.at[pl.ds(NS * RPC, RTAIL)])

        @pl.when(cid == 1)
        def _():
            pltpu.sync_copy(acc.at[pl.ds(sid * RPC, RPC)], o1_hbm.at[pl.ds(sid * RPC, RPC)])

            @pl.when(sid == NS - 1)
            def _():
                pltpu.sync_copy(acc.at[pl.ds(NS * RPC, RTAIL)], o1_hbm.at[pl.ds(NS * RPC, RTAIL)])

    return k(msg, dst, zeros)


def _final_body(o0_ref, o1_ref, x_ref, g_ref, b_ref, y_ref):
    o = o0_ref[...] + o1_ref[...]
    mean = jnp.mean(o, axis=0, keepdims=True)
    var = jnp.mean(o * o, axis=0, keepdims=True) - mean * mean
    inv = lax.rsqrt(var + EPS)
    y = x_ref[...] + (o - mean) * inv * g_ref[...] + b_ref[...]
    y_ref[...] = jnp.maximum(y, 0.0)


def _tc_final(o0, o1, x, g, b):
    full = lambda r: pl.BlockSpec((r, D), lambda: (0, 0))
    return pl.pallas_call(
        _final_body,
        in_specs=[full(N), full(N), full(N), full(1), full(1)],
        out_specs=full(N),
        out_shape=jax.ShapeDtypeStruct((N, D), jnp.float32),
    )(o0, o1, x, g, b)


def kernel(x, edge_index, edge_attr, W1f, b1f, W2f, b2f, W1, b1, W2, b2,
           gamma_i, beta_i, gamma, beta):
    src = edge_index[0]
    dst = edge_index[1]
    bf = jnp.bfloat16
    xs_g, xd_g = _sc_gather(x, src, dst)
    hf, h, st = _tc_mlp(xd_g, xs_g, edge_attr,
                        W1f.T.astype(bf), W2f.T.astype(bf),
                        W1.T.astype(bf), W2.T.astype(bf),
                        b1f[None, :], b2f[None, :], b1[None, :], b2[None, :])
    msg = _tc_msg(hf, h, st, gamma_i[None, :], beta_i[None, :])
    zeros = jnp.zeros((N, D), jnp.float32)
    o0, o1 = _sc_scatter(msg, dst, zeros)
    return _tc_final(o0, o1, x, gamma[None, :], beta[None, :])


# speedup vs baseline: 2.9566x; 1.1577x over previous
import functools

import jax
import jax.numpy as jnp
from jax import lax
from jax.experimental import pallas as pl
from jax.experimental.pallas import tpu as pltpu
from jax.experimental.pallas import tpu_sc as plsc

N = 10000
E = 320000
D = 128

NC = 2
NS = 16
NW = NC * NS

NSLAB = 5
ES = E // NSLAB
EPW = ES // NW
CH = 128
NFULL = (EPW // CH) * CH
TAIL = EPW - NFULL

BE = 1280
GBS = ES // BE
EPS = 1e-5


def _vmesh():
    return plsc.VectorSubcoreMesh(core_axis_name="c", subcore_axis_name="s",
                                  num_cores=NC, num_subcores=NS)


def _sc_gather(x, src, dst):
    @functools.partial(
        pl.kernel,
        out_type=(jax.ShapeDtypeStruct((ES, D), jnp.float32),
                  jax.ShapeDtypeStruct((ES, D), jnp.float32)),
        mesh=_vmesh(),
        scratch_types=[
            pltpu.VMEM((CH,), jnp.int32), pltpu.VMEM((CH,), jnp.int32),
            pltpu.VMEM((CH, D), jnp.float32), pltpu.VMEM((CH, D), jnp.float32),
            pltpu.VMEM((TAIL,), jnp.int32), pltpu.VMEM((TAIL,), jnp.int32),
            pltpu.VMEM((TAIL, D), jnp.float32), pltpu.VMEM((TAIL, D), jnp.float32),
            pltpu.SemaphoreType.DMA, pltpu.SemaphoreType.DMA,
        ],
    )
    def k(x_hbm, src_hbm, dst_hbm, xs_hbm, xd_hbm,
          si, di, srows, drows, sit, dit, srowst, drowst, sem_s, sem_d):
        wid = lax.axis_index("s") * NC + lax.axis_index("c")
        base = wid * EPW

        @pl.loop(0, NFULL, step=CH)
        def _(off):
            b = base + off
            pltpu.sync_copy(src_hbm.at[pl.ds(b, CH)], si)
            pltpu.sync_copy(dst_hbm.at[pl.ds(b, CH)], di)
            cs = pltpu.async_copy(x_hbm.at[si], srows, sem_s)
            cd = pltpu.async_copy(x_hbm.at[di], drows, sem_d)
            cs.wait()
            cd.wait()
            pltpu.sync_copy(srows, xs_hbm.at[pl.ds(b, CH)])
            pltpu.sync_copy(drows, xd_hbm.at[pl.ds(b, CH)])

        bt = base + NFULL
        pltpu.sync_copy(src_hbm.at[pl.ds(bt, TAIL)], sit)
        pltpu.sync_copy(dst_hbm.at[pl.ds(bt, TAIL)], dit)
        cs = pltpu.async_copy(x_hbm.at[sit], srowst, sem_s)
        cd = pltpu.async_copy(x_hbm.at[dit], drowst, sem_d)
        cs.wait()
        cd.wait()
        pltpu.sync_copy(srowst, xs_hbm.at[pl.ds(bt, TAIL)])
        pltpu.sync_copy(drowst, xd_hbm.at[pl.ds(bt, TAIL)])

    return k(x, src, dst)


def _silu(v):
    return v * jax.nn.sigmoid(v)


def _mlp_body(xd_ref, xs_ref, ea_ref, w1f_ref, w2f_ref, w1_ref, w2_ref,
              b1f_ref, b2f_ref, b1_ref, b2_ref, hf_ref, h_ref, st_ref):
    i = pl.program_id(0)
    xd = xd_ref[...].astype(jnp.bfloat16)
    xs = xs_ref[...].astype(jnp.bfloat16)
    ea = ea_ref[...].astype(jnp.bfloat16)
    w1f = w1f_ref[...]
    uf = (jnp.dot(xd, w1f[0:D], preferred_element_type=jnp.float32)
          + jnp.dot(xs, w1f[D:2 * D], preferred_element_type=jnp.float32)
          + jnp.dot(ea, w1f[2 * D:3 * D], preferred_element_type=jnp.float32)
          + b1f_ref[...])
    hf = jnp.dot(_silu(uf).astype(jnp.bfloat16), w2f_ref[...],
                 preferred_element_type=jnp.float32) + b2f_ref[...]
    w1 = w1_ref[...]
    u = (jnp.dot(xd, w1[0:D], preferred_element_type=jnp.float32)
         + jnp.dot(xs, w1[D:2 * D], preferred_element_type=jnp.float32)
         + jnp.dot(ea, w1[2 * D:3 * D], preferred_element_type=jnp.float32)
         + b1_ref[...])
    h = jnp.dot(_silu(u).astype(jnp.bfloat16), w2_ref[...],
                preferred_element_type=jnp.float32) + b2_ref[...]
    hf_ref[...] = hf.astype(jnp.bfloat16)
    h_ref[...] = h.astype(jnp.bfloat16)

    s1 = jnp.sum(hf, axis=0, keepdims=True)
    s2 = jnp.sum(hf * hf, axis=0, keepdims=True)
    upd = jnp.concatenate([s1, s2, jnp.zeros((6, D), jnp.float32)], axis=0)

    @pl.when(i == 0)
    def _():
        st_ref[...] = jnp.zeros_like(st_ref)

    st_ref[...] += upd


def _tc_mlp(xd, xs, ea, w1ft, w2ft, w1t, w2t, b1f, b2f, b1, b2):
    blk = lambda: pl.BlockSpec((BE, D), lambda i: (i, 0))
    full = lambda r: pl.BlockSpec((r, D), lambda i: (0, 0))
    return pl.pallas_call(
        _mlp_body,
        grid=(GBS,),
        in_specs=[blk(), blk(), blk(),
                  full(3 * D), full(D), full(3 * D), full(D),
                  full(1), full(1), full(1), full(1)],
        out_specs=[blk(), blk(), full(8)],
        out_shape=(jax.ShapeDtypeStruct((ES, D), jnp.bfloat16),
                   jax.ShapeDtypeStruct((ES, D), jnp.bfloat16),
                   jax.ShapeDtypeStruct((8, D), jnp.float32)),
    )(xd, xs, ea, w1ft, w2ft, w1t, w2t, b1f, b2f, b1, b2)


def _msg_body(hf_ref, h_ref, st_ref, gi_ref, bi_ref, msg_ref):
    st = jnp.sum(st_ref[...], axis=0)
    mean = st[0:1] * (1.0 / E)
    var = st[1:2] * (1.0 / E) - mean * mean
    inv = lax.rsqrt(var + EPS)
    hf = hf_ref[...].astype(jnp.float32)
    score = jax.nn.sigmoid((hf - mean) * inv * gi_ref[...] + bi_ref[...])
    msg_ref[...] = score * h_ref[...].astype(jnp.float32)


def _tc_msg(hf, h, st_all, gi, bi):
    blk = lambda: pl.BlockSpec((BE, D), lambda i: (i, 0))
    return pl.pallas_call(
        _msg_body,
        grid=(GBS,),
        in_specs=[blk(), blk(),
                  pl.BlockSpec((NSLAB, 8, D), lambda i: (0, 0, 0)),
                  pl.BlockSpec((1, D), lambda i: (0, 0)),
                  pl.BlockSpec((1, D), lambda i: (0, 0))],
        out_specs=blk(),
        out_shape=jax.ShapeDtypeStruct((ES, D), jnp.float32),
    )(hf, h, st_all, gi, bi)


def _sc_scatter(msg, dst, init0, init1):
    RPC = 624
    RTAIL = N - RPC * NS

    @functools.partial(
        pl.kernel,
        out_type=(jax.ShapeDtypeStruct((N, D), jnp.float32),
                  jax.ShapeDtypeStruct((N, D), jnp.float32)),
        mesh=_vmesh(),
        scratch_types=[
            pltpu.VMEM((CH,), jnp.int32), pltpu.VMEM((CH, D), jnp.float32),
            pltpu.VMEM((TAIL,), jnp.int32), pltpu.VMEM((TAIL, D), jnp.float32),
            pltpu.VMEM_SHARED((N, D), jnp.float32),
        ],
    )
    def k(msg_hbm, dst_hbm, z0_hbm, z1_hbm, o0_hbm, o1_hbm,
          idx, rows, idxt, rowst, acc):
        cid = lax.axis_index("c")
        sid = lax.axis_index("s")
        wid = sid * NC + cid
        base = wid * EPW

        @pl.when(cid == 0)
        def _():
            pltpu.sync_copy(z0_hbm.at[pl.ds(sid * RPC, RPC)], acc.at[pl.ds(sid * RPC, RPC)])

            @pl.when(sid == NS - 1)
            def _():
                pltpu.sync_copy(z0_hbm.at[pl.ds(NS * RPC, RTAIL)], acc.at[pl.ds(NS * RPC, RTAIL)])

        @pl.when(cid == 1)
        def _():
            pltpu.sync_copy(z1_hbm.at[pl.ds(sid * RPC, RPC)], acc.at[pl.ds(sid * RPC, RPC)])

            @pl.when(sid == NS - 1)
            def _():
                pltpu.sync_copy(z1_hbm.at[pl.ds(NS * RPC, RTAIL)], acc.at[pl.ds(NS * RPC, RTAIL)])

        plsc.subcore_barrier()

        @pl.loop(0, NFULL, step=CH)
        def _(off):
            b = base + off
            pltpu.sync_copy(dst_hbm.at[pl.ds(b, CH)], idx)
            pltpu.sync_copy(msg_hbm.at[pl.ds(b, CH)], rows)
            pltpu.sync_copy(rows, acc.at[idx], add=True)

        bt = base + NFULL
        pltpu.sync_copy(dst_hbm.at[pl.ds(bt, TAIL)], idxt)
        pltpu.sync_copy(msg_hbm.at[pl.ds(bt, TAIL)], rowst)
        pltpu.sync_copy(rowst, acc.at[idxt], add=True)

        plsc.subcore_barrier()

        @pl.when(cid == 0)
        def _():
            pltpu.sync_copy(acc.at[pl.ds(sid * RPC, RPC)], o0_hbm.at[pl.ds(sid * RPC, RPC)])

            @pl.when(sid == NS - 1)
            def _():
                pltpu.sync_copy(acc.at[pl.ds(NS * RPC, RTAIL)], o0_hbm.at[pl.ds(NS * RPC, RTAIL)])

        @pl.when(cid == 1)
        def _():
            pltpu.sync_copy(acc.at[pl.ds(sid * RPC, RPC)], o1_hbm.at[pl.ds(sid * RPC, RPC)])

            @pl.when(sid == NS - 1)
            def _():
                pltpu.sync_copy(acc.at[pl.ds(NS * RPC, RTAIL)], o1_hbm.at[pl.ds(NS * RPC, RTAIL)])

    return k(msg, dst, init0, init1)


def _final_body(*refs):
    part_refs = refs[:-4]
    x_ref, g_ref, b_ref, y_ref = refs[-4:]
    o = part_refs[0][...]
    for r in part_refs[1:]:
        o = o + r[...]
    mean = jnp.mean(o, axis=0, keepdims=True)
    var = jnp.mean(o * o, axis=0, keepdims=True) - mean * mean
    inv = lax.rsqrt(var + EPS)
    y = x_ref[...] + (o - mean) * inv * g_ref[...] + b_ref[...]
    y_ref[...] = jnp.maximum(y, 0.0)


def _tc_final(partials, x, g, b):
    nd = lambda: pl.BlockSpec((N, D), lambda: (0, 0))
    return pl.pallas_call(
        _final_body,
        in_specs=[nd() for _ in partials] + [nd(),
                  pl.BlockSpec((1, D), lambda: (0, 0)),
                  pl.BlockSpec((1, D), lambda: (0, 0))],
        out_specs=nd(),
        out_shape=jax.ShapeDtypeStruct((N, D), jnp.float32),
    )(*partials, x, g, b)


def kernel(x, edge_index, edge_attr, W1f, b1f, W2f, b2f, W1, b1, W2, b2,
           gamma_i, beta_i, gamma, beta):
    bf = jnp.bfloat16
    src = edge_index[0]
    dst = edge_index[1]
    w1ft = W1f.T.astype(bf)
    w2ft = W2f.T.astype(bf)
    w1t = W1.T.astype(bf)
    w2t = W2.T.astype(bf)

    gathered = []
    for s in range(NSLAB):
        sl = slice(s * ES, (s + 1) * ES)
        gathered.append(_sc_gather(x, src[sl], dst[sl]))

    mlp_out = []
    for s in range(NSLAB):
        sl = slice(s * ES, (s + 1) * ES)
        xs_g, xd_g = gathered[s]
        mlp_out.append(_tc_mlp(xd_g, xs_g, edge_attr[sl],
                               w1ft, w2ft, w1t, w2t,
                               b1f[None, :], b2f[None, :], b1[None, :], b2[None, :]))

    st_all = jnp.stack([m[2] for m in mlp_out], axis=0)

    o0 = jnp.zeros((N, D), jnp.float32)
    o1 = jnp.zeros((N, D), jnp.float32)
    for s in range(NSLAB):
        sl = slice(s * ES, (s + 1) * ES)
        hf, h, _ = mlp_out[s]
        msg = _tc_msg(hf, h, st_all, gamma_i[None, :], beta_i[None, :])
        o0, o1 = _sc_scatter(msg, dst[sl], o0, o1)

    return _tc_final([o0, o1], x, gamma[None, :], beta[None, :])
